# baseline (speedup 1.0000x reference)
import functools

import jax
import jax.numpy as jnp
from jax import lax
from jax.experimental import pallas as pl
from jax.experimental.pallas import tpu as pltpu
from jax.experimental.pallas import tpu_sc as plsc


def _order_map_sc(B, N, D, I):
    info = plsc.get_sparse_core_info()
    NC, NS, L = info.num_cores, info.num_subcores, info.num_lanes
    NW = NC * NS
    TPB = N // 128
    DB = D // 8
    K = D // 2
    CI = L
    CW = CI * K
    n_chunks = I // CI
    n_rounds = 4
    rc = n_chunks // n_rounds
    assert NW == 2 * B and D == 64 and N % 128 == 0 and L == 16
    assert I % CI == 0 and n_chunks % n_rounds == 0

    mesh = plsc.VectorSubcoreMesh(core_axis_name="c", subcore_axis_name="s")

    @functools.partial(
        pl.kernel,
        mesh=mesh,
        out_type=jax.ShapeDtypeStruct((NW * D, 128), jnp.float32),
        scratch_types=[
            pltpu.VMEM((I,), jnp.int32),
            pltpu.VMEM((I,), jnp.int32),
            pltpu.VMEM((I,), jnp.int32),
            pltpu.VMEM((K,), jnp.int32),
            pltpu.VMEM((n_chunks, CW), jnp.int32),
            pltpu.VMEM((rc, CW, 16), jnp.float32),
            pltpu.VMEM((rc, CW, 16), jnp.float32),
            pltpu.VMEM((2 * K, 128), jnp.float32),
            pltpu.SemaphoreType.DMA,
            pltpu.SemaphoreType.DMA,
        ],
        compiler_params=pltpu.CompilerParams(
            needs_layout_passes=False, use_tc_tiling_on_sc=False),
    )
    def gather_kernel(z_hbm, idx_hbm, out_hbm, idx_v, g_v, lane_v, pat_v,
                      ids_v, grp0_v, grp1_v, out_v, sem0, sem1):
        wid = lax.axis_index("s") * NC + lax.axis_index("c")
        b = lax.shift_right_logical(wid, 1)
        db0 = lax.bitwise_and(wid, 1) * (DB // 2)
        lanes = lax.iota(jnp.int32, L)

        pltpu.sync_copy(idx_hbm, idx_v)
        for j in range(I // L):
            v = idx_v[pl.ds(j * L, L)]
            c = jnp.minimum(jnp.maximum(v, 0), N - 1)
            g_v[pl.ds(j * L, L)] = (
                lax.shift_right_logical(c, 7) * 64
                + lax.bitwise_and(lax.shift_right_logical(c, 4), 7))
            lane_v[pl.ds(j * L, L)] = lax.bitwise_and(c, 15)

        for j in range(K // L):
            k16 = lanes + j * L
            rb16 = b * DB + db0 + lax.shift_right_logical(k16, 3)
            pat_v[pl.ds(j * L, L)] = (
                rb16 * (TPB * 64) + lax.bitwise_and(k16, 7) * 8)

        def build_ids(cidx, carry):
            for j in range(CW // L):
                i16 = jnp.full((L,), j * L // K, jnp.int32) + cidx * CI
                g16 = plsc.load_gather(g_v, [i16])
                ids_v[cidx, pl.ds(j * L, L)] = (
                    pat_v[pl.ds((j % (K // L)) * L, L)] + g16)
            return carry
        lax.fori_loop(0, n_chunks, build_ids, 0)

        grps = (grp0_v, grp1_v)
        sems = (sem0, sem1)

        def issue_round(r):
            for jj in range(rc):
                pltpu.async_copy(z_hbm.at[ids_v.at[r * rc + jj]],
                                 grps[r % 2].at[jj], sems[r % 2])

        def drain_round(r):
            for jj in range(rc):
                pltpu.make_async_copy(z_hbm.at[ids_v.at[r * rc + jj]],
                                      grps[r % 2].at[jj],
                                      sems[r % 2]).wait()

        def select_round(r):
            grp = grps[r % 2]

            def body(jj, carry):
                i16 = (r * rc + jj) * CI + lanes
                cols16 = plsc.load_gather(lane_v, [i16])
                row_hi16 = lax.shift_right_logical(i16, 7) * 8
                col16 = lax.bitwise_and(i16, 127)
                jj16 = jnp.full((L,), 0, jnp.int32) + jj
                for kk in range(K):
                    r16 = lanes * K + kk
                    vals = plsc.load_gather(grp, [jj16, r16, cols16])
                    row16 = row_hi16 + ((kk >> 3) * 16 + (kk & 7))
                    plsc.store_scatter(out_v, [row16, col16], vals)
                return carry
            lax.fori_loop(0, rc, body, 0)

        issue_round(0)
        for r in range(n_rounds):
            if r + 1 < n_rounds:
                issue_round(r + 1)
            drain_round(r)
            select_round(r)

        pltpu.sync_copy(out_v, out_hbm.at[pl.ds(wid * (2 * K), 2 * K)])

    return gather_kernel


def kernel(x, indices):
    B, N, D = x.shape
    I = indices.shape[0]
    NW = 2 * B
    xf = x.astype(jnp.float32)
    xt = jnp.transpose(xf, (0, 2, 1))
    z = (xt.reshape(B * D // 8, 8, N // 128, 128)
           .transpose(0, 2, 1, 3)
           .reshape(B * D * N // 16, 16))
    out_p = _order_map_sc(B, N, D, I)(z, indices)
    out_t = (out_p.reshape(B * D // 8, 2, 8, 128)
                  .transpose(0, 2, 1, 3)
                  .reshape(B * D, I))
    return jnp.transpose(out_t.reshape(B, D, I), (0, 2, 1))
